# baseline (speedup 1.0000x reference)
import functools
import jax
import jax.numpy as jnp
import numpy as np
from jax.experimental import pallas as pl

N = 10000
E = 320000
T = 1280000
B = 8
NR = 64
NS = 7
NB = 3
EA = 128
EE = 128
ET = 32
ER = 16
CUT = 6.0

_CHEB = np.array([
    [1, 0, 0, 0, 0, 0, 0],
    [0, 1, 0, 0, 0, 0, 0],
    [-1, 0, 2, 0, 0, 0, 0],
    [0, -3, 0, 4, 0, 0, 0],
    [1, 0, -8, 0, 8, 0, 0],
    [0, 5, 0, -20, 0, 16, 0],
    [-1, 0, 18, 0, -48, 0, 32],
], dtype=np.float32)


def _silu(x):
    return x * jax.nn.sigmoid(x)


_EBLK = 512


def _rbf_body(pv_ref, out_ref):
    vec = pv_ref[:, 0:3]
    d2 = jnp.sum(vec * vec, axis=1, keepdims=True) + 1e-10
    d = jnp.sqrt(d2)
    x = jnp.clip(d / CUT, 0.0, 1.0)
    env = jnp.where(x < 1.0, 1.0 - 10.0 * x**3 + 15.0 * x**4 - 6.0 * x**5, 0.0)
    k = (jax.lax.broadcasted_iota(jnp.int32, (1, NR), 1) + 1).astype(jnp.float32)
    out_ref[...] = env * jnp.sin(k * (jnp.pi * x)) / d


def _rbf_pallas(vec_pad):
    return pl.pallas_call(
        _rbf_body,
        grid=(E // _EBLK,),
        in_specs=[pl.BlockSpec((_EBLK, 128), lambda i: (i, 0))],
        out_specs=pl.BlockSpec((_EBLK, NR), lambda i: (i, 0)),
        out_shape=jax.ShapeDtypeStruct((E, NR), jnp.float32),
    )(vec_pad)


def kernel(pos, atomic_numbers, edge_index, id3_ba, id3_ca, batch, emb_table,
           W_edge, W_rbf3, W_cbf3, W_rbf_h, W_rbf_out, W_h_gate, W_out_gate,
           W_dba, W_rbf_gate, W_down, W_c2t, W_up, W_atom, W_out):
    src = edge_index[0]
    dst = edge_index[1]
    vec = pos[dst] - pos[src]
    d = jnp.sqrt(jnp.sum(vec * vec, axis=1) + 1e-10)
    V = vec / d[:, None]

    vec_pad = jnp.zeros((E, 128), jnp.float32).at[:, 0:3].set(vec)
    rbf = _rbf_pallas(vec_pad)

    cos_t = jnp.clip(jnp.sum(V[id3_ba] * V[id3_ca], axis=1), -0.999, 0.999)

    Wgo = W_rbf_out @ W_out_gate
    gate_out = rbf @ Wgo
    A1 = emb_table @ W_edge[:EA]
    A2 = emb_table @ W_edge[EA:2 * EA]
    W3 = W_edge[2 * EA:]
    an_src = atomic_numbers[src]
    an_dst = atomic_numbers[dst]
    m = _silu(A1[an_src] + A2[an_dst] + rbf @ W3)

    e_at = _silu(jax.ops.segment_sum(m * gate_out, dst, num_segments=N)) @ W_out[0]
    inv_sqrt2 = 1.0 / jnp.sqrt(2.0)
    cheb_t = jnp.asarray(_CHEB.T)
    for i in range(NB):
        WGi = W_rbf3 @ W_rbf_gate[i]
        xb = _silu(m @ W_dba[i]) * (rbf @ WGi)
        xd = xb @ W_down[i]
        Ci = cheb_t @ (W_cbf3 @ W_c2t[i])
        g = jnp.broadcast_to(Ci[6], (T, ET))
        for k in range(5, -1, -1):
            g = g * cos_t[:, None] + Ci[k]
        x3 = xd[id3_ba] * g
        agg = jax.ops.segment_sum(x3, id3_ca, num_segments=E)
        m = (m + _silu(agg @ W_up[i])) * inv_sqrt2
        e_at = e_at + _silu(jax.ops.segment_sum(m * gate_out, dst, num_segments=N)) @ W_out[i + 1]
    energy = jax.ops.segment_sum(e_at, batch, num_segments=B)
    return energy


# speedup vs baseline: 4.4187x; 4.4187x over previous
import functools
import jax
import jax.numpy as jnp
import numpy as np
from jax import lax
from jax.experimental import pallas as pl
from jax.experimental.pallas import tpu as pltpu
from jax.experimental.pallas import tpu_sc as plsc

N = 10000
E = 320000
T = 1280000
B = 8
NR = 64
NS = 7
NB = 3
EA = 128
EE = 128
ET = 32
ER = 16
CUT = 6.0

_CHEB = np.array([
    [1, 0, 0, 0, 0, 0, 0],
    [0, 1, 0, 0, 0, 0, 0],
    [-1, 0, 2, 0, 0, 0, 0],
    [0, -3, 0, 4, 0, 0, 0],
    [1, 0, -8, 0, 8, 0, 0],
    [0, 5, 0, -20, 0, 16, 0],
    [-1, 0, 18, 0, -48, 0, 32],
], dtype=np.float32)

_SC_CORES = 2
_SC_SUBCORES = 16
_NW = _SC_CORES * _SC_SUBCORES
_W = 800
_IB = 80
_NBI = _W // _IB


def _silu(x):
    return x * jax.nn.sigmoid(x)


_EBLK = 512


def _geom_body(pv_ref, rbf_ref, v4_ref):
    vec = pv_ref[:, 0:4]
    d2 = jnp.sum(vec * vec, axis=1, keepdims=True) + 1e-10
    d = jnp.sqrt(d2)
    v4_ref[...] = vec / d
    x = jnp.clip(d / CUT, 0.0, 1.0)
    env = jnp.where(x < 1.0, 1.0 - 10.0 * x**3 + 15.0 * x**4 - 6.0 * x**5, 0.0)
    k = (lax.broadcasted_iota(jnp.int32, (1, NR), 1) + 1).astype(jnp.float32)
    rbf_ref[...] = env * jnp.sin(k * (jnp.pi * x)) / d


def _geom_pallas(vec_pad):
    return pl.pallas_call(
        _geom_body,
        grid=(E // _EBLK,),
        in_specs=[pl.BlockSpec((_EBLK, 128), lambda i: (i, 0))],
        out_specs=[pl.BlockSpec((_EBLK, NR), lambda i: (i, 0)),
                   pl.BlockSpec((_EBLK, 4), lambda i: (i, 0))],
        out_shape=[jax.ShapeDtypeStruct((E, NR), jnp.float32),
                   jax.ShapeDtypeStruct((E, 4), jnp.float32)],
    )(vec_pad)


def _cos_sc(vx, vy, vz, ba, ca):
    chunk = T // _NW
    nwin = chunk // _W
    mesh = plsc.VectorSubcoreMesh(core_axis_name="c", subcore_axis_name="s")

    @functools.partial(
        pl.kernel,
        out_type=jax.ShapeDtypeStruct((T,), jnp.float32),
        mesh=mesh,
        compiler_params=pltpu.CompilerParams(use_tc_tiling_on_sc=False),
        scratch_types=[
            pltpu.VMEM((_W,), jnp.int32),
            pltpu.VMEM((_W,), jnp.int32),
            [pltpu.VMEM((_W,), jnp.float32) for _ in range(6)],
            pltpu.VMEM((_W,), jnp.float32),
            pltpu.SemaphoreType.DMA,
        ],
    )
    def k(vx_h, vy_h, vz_h, ba_h, ca_h, cos_h, ba_v, ca_v, comps, cos_v, sem):
        wid = lax.axis_index("s") * _SC_CORES + lax.axis_index("c")
        base = wid * chunk
        ax, ay, az, bx, by, bz = comps

        def win(w, carry):
            off = base + w * _W
            pltpu.sync_copy(ba_h.at[pl.ds(off, _W)], ba_v)
            pltpu.sync_copy(ca_h.at[pl.ds(off, _W)], ca_v)

            def batch(j, c2):
                s = pl.ds(j * _IB, _IB)
                descs = [
                    pltpu.async_copy(vx_h.at[ba_v.at[s]], ax.at[s], sem),
                    pltpu.async_copy(vy_h.at[ba_v.at[s]], ay.at[s], sem),
                    pltpu.async_copy(vz_h.at[ba_v.at[s]], az.at[s], sem),
                    pltpu.async_copy(vx_h.at[ca_v.at[s]], bx.at[s], sem),
                    pltpu.async_copy(vy_h.at[ca_v.at[s]], by.at[s], sem),
                    pltpu.async_copy(vz_h.at[ca_v.at[s]], bz.at[s], sem),
                ]
                for dsc in descs:
                    dsc.wait()
                return c2

            lax.fori_loop(0, _NBI, batch, 0)
            for g in range(_W // 16):
                s = pl.ds(g * 16, 16)
                acc = ax[s] * bx[s] + ay[s] * by[s] + az[s] * bz[s]
                cos_v[s] = jnp.clip(acc, -0.999, 0.999)
            pltpu.sync_copy(cos_v, cos_h.at[pl.ds(off, _W)])
            return carry

        lax.fori_loop(0, nwin, win, 0)

    return k(vx, vy, vz, ba, ca)


def _gather_sc(xd, ba):
    chunk = T // _NW
    nwin = chunk // _W
    mesh = plsc.VectorSubcoreMesh(core_axis_name="c", subcore_axis_name="s")

    @functools.partial(
        pl.kernel,
        out_type=jax.ShapeDtypeStruct((T, ET), jnp.float32),
        mesh=mesh,
        compiler_params=pltpu.CompilerParams(use_tc_tiling_on_sc=False),
        scratch_types=[
            pltpu.VMEM((_W,), jnp.int32),
            pltpu.VMEM((_W, ET), jnp.float32),
            pltpu.SemaphoreType.DMA,
        ],
    )
    def k(xd_h, ba_h, x2_h, ba_v, rows, sem):
        wid = lax.axis_index("s") * _SC_CORES + lax.axis_index("c")
        base = wid * chunk

        def win(w, carry):
            off = base + w * _W
            pltpu.sync_copy(ba_h.at[pl.ds(off, _W)], ba_v)
            descs = []
            for j in range(_NBI):
                s = pl.ds(j * _IB, _IB)
                descs.append(pltpu.async_copy(xd_h.at[ba_v.at[s]], rows.at[s], sem))
            for dsc in descs:
                dsc.wait()
            pltpu.sync_copy(rows, x2_h.at[pl.ds(off, _W)])
            return carry

        lax.fori_loop(0, nwin, win, 0)

    return k(xd, ba)


def kernel(pos, atomic_numbers, edge_index, id3_ba, id3_ca, batch, emb_table,
           W_edge, W_rbf3, W_cbf3, W_rbf_h, W_rbf_out, W_h_gate, W_out_gate,
           W_dba, W_rbf_gate, W_down, W_c2t, W_up, W_atom, W_out):
    src = edge_index[0]
    dst = edge_index[1]
    vec = pos[dst] - pos[src]

    vec_pad = jnp.zeros((E, 128), jnp.float32).at[:, 0:3].set(vec)
    rbf, V4 = _geom_pallas(vec_pad)

    cos_t = _cos_sc(V4[:, 0], V4[:, 1], V4[:, 2], id3_ba, id3_ca)

    gate_out = (rbf @ W_rbf_out) @ W_out_gate
    rbf3 = rbf @ W_rbf3
    A1 = emb_table @ W_edge[:EA]
    A2 = emb_table @ W_edge[EA:2 * EA]
    W3 = W_edge[2 * EA:]
    an_src = atomic_numbers[src]
    an_dst = atomic_numbers[dst]
    m = _silu(A1[an_src] + A2[an_dst] + rbf @ W3)

    e_at = _silu(jax.ops.segment_sum(m * gate_out, dst, num_segments=N)) @ W_out[0]
    inv_sqrt2 = 1.0 / jnp.sqrt(2.0)
    cheb_t = jnp.asarray(_CHEB.T)
    for i in range(NB):
        xb = _silu(m @ W_dba[i]) * (rbf3 @ W_rbf_gate[i])
        xd = xb @ W_down[i]
        Ci = cheb_t @ (W_cbf3 @ W_c2t[i])
        x2 = _gather_sc(xd, id3_ba)
        g = jnp.broadcast_to(Ci[6], (T, ET))
        for k in range(5, -1, -1):
            g = g * cos_t[:, None] + Ci[k]
        x3 = x2 * g
        agg = jax.ops.segment_sum(x3, id3_ca, num_segments=E)
        m = (m + _silu(agg @ W_up[i])) * inv_sqrt2
        e_at = e_at + _silu(jax.ops.segment_sum(m * gate_out, dst, num_segments=N)) @ W_out[i + 1]
    energy = jax.ops.segment_sum(e_at, batch, num_segments=B)
    return energy
